# baseline (speedup 1.0000x reference)
import functools

import jax
import jax.numpy as jnp
from jax import lax
from jax.experimental import pallas as pl
from jax.experimental.pallas import tpu as pltpu
from jax.experimental.pallas import tpu_sc as plsc

_NUM_CORES = 2
_NUM_SUBCORES = 16
_NW = _NUM_CORES * _NUM_SUBCORES
_CHUNK = 128
_NBUF = 5
_TBUF = 3


def _sc_transpose(data_t, aux):
    c_dim, v = data_t.shape
    nblk = v // _CHUNK
    full = nblk * _CHUNK
    pairs = full // 2 + aux.shape[0]
    mesh = plsc.VectorSubcoreMesh(
        core_axis_name="c", subcore_axis_name="s",
        num_cores=_NUM_CORES, num_subcores=_NUM_SUBCORES)

    base_n = nblk // _NW
    extra = nblk - base_n * _NW

    @functools.partial(
        pl.kernel,
        out_type=jax.ShapeDtypeStruct((pairs, _CHUNK), jnp.float32),
        mesh=mesh,
        compiler_params=pltpu.CompilerParams(
            use_tc_tiling_on_sc=True, needs_layout_passes=False),
        scratch_types=[
            pltpu.VMEM((_TBUF, c_dim, _CHUNK), jnp.float32),
            pltpu.VMEM((_TBUF, c_dim, _CHUNK), jnp.float32),
            pltpu.VMEM((aux.shape[0], _CHUNK), jnp.float32),
            [pltpu.SemaphoreType.DMA] * _TBUF,
            [pltpu.SemaphoreType.DMA] * _TBUF,
        ],
    )
    def body(dt_hbm, aux_hbm, out_hbm, in_v, tr_v, aux_v, isems, osems):
        wid = lax.axis_index("s") * _NUM_CORES + lax.axis_index("c")
        start = wid * base_n + jnp.minimum(wid, extra)
        count = base_n + jnp.where(wid < extra, 1, 0)

        @pl.when(wid == 0)
        def _():
            pltpu.sync_copy(aux_hbm, aux_v)
            pltpu.sync_copy(aux_v, out_hbm.at[pl.ds(full // 2, aux.shape[0])])

        def in_start(k, b):
            pltpu.async_copy(
                dt_hbm.at[:, pl.ds((start + k) * _CHUNK, _CHUNK)],
                in_v.at[b], isems[b])

        def in_wait(k, b):
            pltpu.make_async_copy(
                dt_hbm.at[:, pl.ds((start + k) * _CHUNK, _CHUNK)],
                in_v.at[b], isems[b]).wait()

        def out_start(k, b):
            pltpu.async_copy(
                tr_v.at[b], out_hbm.at[pl.ds((start + k) * (_CHUNK // 2),
                                             _CHUNK // 2)], osems[b])

        def out_wait(k, b):
            pltpu.make_async_copy(
                tr_v.at[b], out_hbm.at[pl.ds((start + k) * (_CHUNK // 2),
                                             _CHUNK // 2)], osems[b]).wait()

        for b in range(_TBUF):
            @pl.when(b < count)
            def _():
                in_start(b, b)

        def group(g, carry):
            for b in range(_TBUF):
                k = g * _TBUF + b

                @pl.when(k < count)
                def _():
                    in_wait(k, b)

                    @pl.when(k >= _TBUF)
                    def _():
                        out_wait(k - _TBUF, b)

                    def col(c, carry2):
                        lanes = jnp.arange(16, dtype=jnp.int32)
                        half = lax.shift_right_logical(lanes, 1)
                        parity64 = lax.shift_left(lanes & 1, 6)
                        cidx = c + parity64
                        for gg in range(8):
                            vec = in_v[b, c, pl.ds(16 * gg, 16)]
                            plsc.store_scatter(
                                tr_v.at[b], [8 * gg + half, cidx], vec)
                        return carry2

                    lax.fori_loop(0, c_dim, col, 0)
                    out_start(k, b)

                    kn = k + _TBUF

                    @pl.when(kn < count)
                    def _():
                        in_start(kn, b)

            return carry

        ngroups = (base_n + 1 + _TBUF - 1) // _TBUF
        lax.fori_loop(0, ngroups, group, 0)
        for b in range(_TBUF):
            k_last = count - _TBUF + b

            @pl.when(k_last >= 0)
            def _():
                out_wait(k_last, b)

    return body(data_t, aux)


def _sc_gather(table, idx3d):
    nw, per_w, chunk = idx3d.shape
    d = table.shape[1]
    mesh = plsc.VectorSubcoreMesh(
        core_axis_name="c", subcore_axis_name="s",
        num_cores=_NUM_CORES, num_subcores=_NUM_SUBCORES)

    nbuf = _NBUF
    assert per_w % nbuf == 0 and per_w > nbuf

    @functools.partial(
        pl.kernel,
        out_type=jax.ShapeDtypeStruct((nw * per_w * chunk, d), jnp.float32),
        mesh=mesh,
        compiler_params=pltpu.CompilerParams(use_tc_tiling_on_sc=True),
        scratch_types=[
            pltpu.VMEM((per_w, chunk), jnp.int32),
            pltpu.VMEM((nbuf, chunk, d), jnp.float32),
            [pltpu.SemaphoreType.DMA] * nbuf,
            [pltpu.SemaphoreType.DMA] * nbuf,
        ],
    )
    def body(table_hbm, idx_hbm, out_hbm, idx_v, rows_v, gsems, ssems):
        wid = lax.axis_index("s") * _NUM_CORES + lax.axis_index("c")
        base = wid * per_w
        pltpu.sync_copy(idx_hbm.at[wid], idx_v)

        def gather_start(j, b):
            pltpu.async_copy(table_hbm.at[idx_v.at[j]], rows_v.at[b], gsems[b])

        def gather_wait(j, b):
            pltpu.make_async_copy(
                table_hbm.at[idx_v.at[j]], rows_v.at[b], gsems[b]).wait()

        def store_start(j, b):
            pltpu.async_copy(
                rows_v.at[b], out_hbm.at[pl.ds((base + j) * chunk, chunk)],
                ssems[b])

        def store_wait(j, b):
            pltpu.make_async_copy(
                rows_v.at[b], out_hbm.at[pl.ds((base + j) * chunk, chunk)],
                ssems[b]).wait()

        for b in range(nbuf):
            gather_start(b, b)

        def outer(g, carry):
            j0 = g * nbuf
            for b in range(nbuf):
                gather_wait(j0 + b, b)
                store_start(j0 + b, b)
            for b in range(nbuf):
                jn = j0 + nbuf + b

                @pl.when(jn < per_w)
                def _():
                    store_wait(j0 + b, b)
                    gather_start(jn, b)

            return carry

        lax.fori_loop(0, per_w // nbuf, outer, 0)
        for b in range(nbuf):
            store_wait(per_w - nbuf + b, b)

    return body(table, idx3d)


def kernel(data, indices, axis):
    v, d = data.shape
    idx = indices.reshape(-1).astype(jnp.int32)
    idx = idx + jnp.asarray(axis, dtype=jnp.int32)
    idx = jnp.where(idx < 0, idx + v, idx)

    full = (v // _CHUNK) * _CHUNK
    aux = data[full:].reshape(-1, 2 * d)
    table = _sc_transpose(jnp.swapaxes(data, 0, 1), aux)

    pair_idx = lax.shift_right_logical(idx, 1)
    per_w = idx.shape[0] // (_NW * _CHUNK)
    idx3d = pair_idx.reshape(_NW, per_w, _CHUNK)
    pairs = _sc_gather(table, idx3d)
    odd = (idx & 1)[:, None] == 1
    out = jnp.where(odd, pairs[:, d:], pairs[:, :d])
    return out.reshape(indices.shape + (d,))


# speedup vs baseline: 2.0483x; 2.0483x over previous
import functools

import jax
import jax.numpy as jnp
from jax import lax
from jax.experimental import pallas as pl
from jax.experimental.pallas import tpu as pltpu
from jax.experimental.pallas import tpu_sc as plsc

_NUM_CORES = 2
_NUM_SUBCORES = 16
_NW = _NUM_CORES * _NUM_SUBCORES
_CHUNK = 128
_NBUF = 5


def _sc_gather(data, idx3d):
    nw, per_w, chunk = idx3d.shape
    d = data.shape[1]
    mesh = plsc.VectorSubcoreMesh(
        core_axis_name="c", subcore_axis_name="s",
        num_cores=_NUM_CORES, num_subcores=_NUM_SUBCORES)

    nbuf = _NBUF
    assert per_w % nbuf == 0 and per_w > nbuf

    @functools.partial(
        pl.kernel,
        out_type=jax.ShapeDtypeStruct((nw * per_w * chunk, d), jnp.float32),
        mesh=mesh,
        compiler_params=pltpu.CompilerParams(use_tc_tiling_on_sc=False),
        scratch_types=[
            pltpu.VMEM((per_w, chunk), jnp.int32),
            pltpu.VMEM((nbuf, chunk, d), jnp.float32),
            [pltpu.SemaphoreType.DMA] * nbuf,
            [pltpu.SemaphoreType.DMA] * nbuf,
        ],
    )
    def body(data_hbm, idx_hbm, out_hbm, idx_v, rows_v, gsems, ssems):
        wid = lax.axis_index("s") * _NUM_CORES + lax.axis_index("c")
        base = wid * per_w
        pltpu.sync_copy(idx_hbm.at[wid], idx_v)

        def gather_start(j, b):
            pltpu.async_copy(data_hbm.at[idx_v.at[j]], rows_v.at[b], gsems[b])

        def gather_wait(j, b):
            pltpu.make_async_copy(
                data_hbm.at[idx_v.at[j]], rows_v.at[b], gsems[b]).wait()

        def store_start(j, b):
            pltpu.async_copy(
                rows_v.at[b], out_hbm.at[pl.ds((base + j) * chunk, chunk)],
                ssems[b])

        def store_wait(j, b):
            pltpu.make_async_copy(
                rows_v.at[b], out_hbm.at[pl.ds((base + j) * chunk, chunk)],
                ssems[b]).wait()

        for b in range(nbuf):
            gather_start(b, b)

        def outer(g, carry):
            j0 = g * nbuf
            for b in range(nbuf):
                gather_wait(j0 + b, b)
                store_start(j0 + b, b)
            for b in range(nbuf):
                jn = j0 + nbuf + b

                @pl.when(jn < per_w)
                def _():
                    store_wait(j0 + b, b)
                    gather_start(jn, b)

            return carry

        lax.fori_loop(0, per_w // nbuf, outer, 0)
        for b in range(nbuf):
            store_wait(per_w - nbuf + b, b)

    return body(data, idx3d)


def kernel(data, indices, axis):
    v, d = data.shape
    idx = indices.reshape(-1).astype(jnp.int32)
    idx = idx + jnp.asarray(axis, dtype=jnp.int32)
    idx = jnp.where(idx < 0, idx + v, idx)
    per_w = idx.shape[0] // (_NW * _CHUNK)
    idx3d = idx.reshape(_NW, per_w, _CHUNK)
    out = _sc_gather(data, idx3d)
    return out.reshape(indices.shape + (d,))
